# Initial kernel scaffold; baseline (speedup 1.0000x reference)
import jax
import jax.numpy as jnp
from jax.experimental import pallas as pl


def kernel(X, edge_index, edge_weight, skip, H, C, Wx, bx, Wh, bh, wc, bg, ln_h_g, ln_h_b, ln_c_g, ln_c_b, ln_o_g, ln_o_b, fc1_w, fc1_b, fc2_w, fc2_b):
    raise NotImplementedError("write your pallas kernel here")


# speedup vs baseline: 7.4943x; 7.4943x over previous
import functools

import jax
import jax.numpy as jnp
from jax import lax
from jax.experimental import pallas as pl
from jax.experimental.pallas import tpu as pltpu
from jax.experimental.pallas import tpu_sc as plsc

N = 10000
N_PAD = 10240
F = 128
E = 320000
E_PAD = 327680
SUPER = 10
SCH = 16
CE = 128
NODES_PER_TILE = N_PAD // 16
RED = 128


def _sc_propagate_body(src_hbm, dst_hbm, w_hbm, xh_hbm, out_hbm,
                       src_sc, dst_sc, w_sc, idx_sc, dinv, tmp, dslice, normc,
                       rows, acc, partials, sdinv, sem):
    cid = lax.axis_index("c")
    sid = lax.axis_index("s")
    zeros16f = jnp.zeros((16,), jnp.float32)

    @pl.loop(0, N_PAD // 16)
    def _zero_deg(i):
        dinv[pl.ds(i * 16, 16)] = zeros16f

    @pl.loop(0, SUPER)
    def _deg_super(sj):
        pltpu.sync_copy(dst_hbm.at[sid, sj], dst_sc)
        pltpu.sync_copy(w_hbm.at[sid, sj], w_sc)

        @pl.loop(0, SCH)
        def _deg(j):
            for k in range(8):
                dv = dst_sc[j, pl.ds(k * 16, 16)]
                wv = w_sc[j, pl.ds(k * 16, 16)]
                plsc.addupdate_scatter(dinv, [dv], wv)

    pltpu.sync_copy(dinv, partials.at[sid])
    plsc.subcore_barrier()

    @pl.loop(0, NODES_PER_TILE // RED)
    def _red(b):
        pltpu.sync_copy(
            partials.at[:, pl.ds(sid * NODES_PER_TILE + b * RED, RED)], tmp)

        @pl.loop(0, RED // 16)
        def _rsqrt(i):
            s = tmp[0, pl.ds(i * 16, 16)]
            for k in range(1, 16):
                s = s + tmp[k, pl.ds(i * 16, 16)]
            bits = plsc.bitcast(s, jnp.int32)
            y = plsc.bitcast(jnp.int32(0x5F3759DF) - (bits >> 1), jnp.float32)
            for _ in range(4):
                y = y * (1.5 - 0.5 * s * y * y)
            dslice[pl.ds(b * RED + i * 16, 16)] = jnp.where(s > 0.0, y, 0.0)

    pltpu.sync_copy(dslice, sdinv.at[pl.ds(sid * NODES_PER_TILE, NODES_PER_TILE)])

    @pl.loop(0, CE)
    def _zero_rows(r):
        for k in range(8):
            rows[r, pl.ds(k * 16, 16)] = zeros16f

    for b in range(NODES_PER_TILE // CE):
        pltpu.sync_copy(rows, acc.at[pl.ds(sid * NODES_PER_TILE + b * CE, CE), :])

    plsc.subcore_barrier()
    pltpu.sync_copy(sdinv, dinv)

    @pl.loop(0, SUPER)
    def _main_super(sj):
        pltpu.sync_copy(src_hbm.at[sid, sj], src_sc)
        pltpu.sync_copy(dst_hbm.at[sid, sj], dst_sc)
        pltpu.sync_copy(w_hbm.at[sid, sj], w_sc)

        @pl.loop(0, SCH)
        def _main(j):
            for k in range(8):
                sv = src_sc[j, pl.ds(k * 16, 16)]
                dv = dst_sc[j, pl.ds(k * 16, 16)]
                wv = w_sc[j, pl.ds(k * 16, 16)]
                nv = -wv * plsc.load_gather(dinv, [sv]) * plsc.load_gather(dinv, [dv])
                normc[pl.ds(k * 16, 16)] = nv
                idx_sc[j, pl.ds(k * 16, 16)] = sv + cid * N

            pltpu.async_copy(xh_hbm.at[idx_sc.at[j]], rows, sem).wait()

            @pl.loop(0, CE)
            def _scale(r):
                nv = plsc.load_gather(normc, [jnp.zeros((16,), jnp.int32) + r])
                for k in range(8):
                    rows[r, pl.ds(k * 16, 16)] = rows[r, pl.ds(k * 16, 16)] * nv

            pltpu.sync_copy(rows, acc.at[dst_sc.at[j]], add=True)

    plsc.subcore_barrier()

    pltpu.sync_copy(acc.at[pl.ds(sid * NODES_PER_TILE, NODES_PER_TILE), :],
                    out_hbm.at[cid, pl.ds(sid * NODES_PER_TILE, NODES_PER_TILE), :])


def _sc_propagate(src4d, dst4d, w4d, xh):
    mesh = plsc.VectorSubcoreMesh(core_axis_name="c", subcore_axis_name="s")
    return pl.kernel(
        _sc_propagate_body,
        out_type=jax.ShapeDtypeStruct((2, N_PAD, F), jnp.float32),
        mesh=mesh,
        scratch_types=[
            pltpu.VMEM((SCH, CE), jnp.int32),
            pltpu.VMEM((SCH, CE), jnp.int32),
            pltpu.VMEM((SCH, CE), jnp.float32),
            pltpu.VMEM((SCH, CE), jnp.int32),
            pltpu.VMEM((N_PAD,), jnp.float32),
            pltpu.VMEM((16, RED), jnp.float32),
            pltpu.VMEM((NODES_PER_TILE,), jnp.float32),
            pltpu.VMEM((CE,), jnp.float32),
            pltpu.VMEM((CE, F), jnp.float32),
            pltpu.VMEM_SHARED((N_PAD, F), jnp.float32),
            pltpu.VMEM_SHARED((16, N_PAD), jnp.float32),
            pltpu.VMEM_SHARED((N_PAD,), jnp.float32),
            pltpu.SemaphoreType.DMA,
        ],
        compiler_params=pltpu.CompilerParams(needs_layout_passes=False),
    )(src4d, dst4d, w4d, xh)


def _tc_body(x_r, px_r, h_r, ph_r, c_r, w0x_r, w1x_r, w0h_r, w1h_r,
             ball_r, wc_r, lnp_r, fc2w_r, fc2b_r, pred_r, hid_r, cell_r):
    x = x_r[...]
    px = px_r[...]
    h = h_r[...]
    ph = ph_r[...]
    c = c_r[...]
    pre = jnp.dot(x, w0x_r[...], preferred_element_type=jnp.float32)
    pre = pre + jnp.dot(px, w1x_r[...], preferred_element_type=jnp.float32)
    pre = pre + jnp.dot(h, w0h_r[...], preferred_element_type=jnp.float32)
    pre = pre + jnp.dot(ph, w1h_r[...], preferred_element_type=jnp.float32)
    pre = pre + ball_r[...]
    wc = wc_r[...]
    ii = pre[:, 0:128]
    ff = pre[:, 128:256]
    gg = pre[:, 256:384]
    oo = pre[:, 384:512]
    i = jax.nn.sigmoid(ii + wc[0:1] * c)
    f = jax.nn.sigmoid(ff + wc[1:2] * c)
    g = jnp.tanh(gg)
    cn = f * c + i * g
    o = jax.nn.sigmoid(oo + wc[2:3] * cn)
    hn = o * jnp.tanh(cn)

    def ln(v, gamma, beta):
        m = jnp.mean(v, axis=-1, keepdims=True)
        d = v - m
        var = jnp.mean(d * d, axis=-1, keepdims=True)
        return d * lax.rsqrt(var + 1e-5) * gamma + beta

    lnp = lnp_r[...]
    hid_r[...] = ln(hn, lnp[0:1], lnp[1:2])
    cell_r[...] = ln(cn, lnp[2:3], lnp[3:4])
    ov = ln(jnp.maximum(hn, 0.0), lnp[4:5], lnp[5:6])
    o2 = jnp.maximum(ov, 0.0)
    pred_r[...] = jax.nn.sigmoid(
        jnp.dot(o2, fc2w_r[...], preferred_element_type=jnp.float32) + fc2b_r[...])


def _tc_call(x, px, h, ph, c, w0x, w1x, w0h, w1h, ball, wc, lnp, fc2w, fc2b):
    R = 1000
    grid = (N // R,)
    row_spec = pl.BlockSpec((R, F), lambda i: (i, 0))
    full = lambda shape: pl.BlockSpec(shape, lambda i: tuple(0 for _ in shape))
    return pl.pallas_call(
        _tc_body,
        grid=grid,
        in_specs=[row_spec, row_spec, row_spec, row_spec, row_spec,
                  full((F, 512)), full((F, 512)), full((F, 512)), full((F, 512)),
                  full((1, 512)), full((3, F)), full((6, F)),
                  full((F, F)), full((1, F))],
        out_specs=[row_spec, row_spec, row_spec],
        out_shape=[jax.ShapeDtypeStruct((N, F), jnp.float32),
                   jax.ShapeDtypeStruct((N, F), jnp.float32),
                   jax.ShapeDtypeStruct((N, F), jnp.float32)],
    )(x, px, h, ph, c, w0x, w1x, w0h, w1h, ball, wc, lnp, fc2w, fc2b)


def kernel(X, edge_index, edge_weight, skip, H, C, Wx, bx, Wh, bh, wc, bg,
           ln_h_g, ln_h_b, ln_c_g, ln_c_b, ln_o_g, ln_o_b,
           fc1_w, fc1_b, fc2_w, fc2_b):
    x = X[0]
    h = H[0]
    c = C[0]
    src = edge_index[0]
    dst = edge_index[1]
    pad = E_PAD - E
    zi = jnp.zeros((pad,), jnp.int32)
    src4d = jnp.concatenate([src, zi]).reshape(16, SUPER, SCH, CE)
    dst4d = jnp.concatenate([dst, zi]).reshape(16, SUPER, SCH, CE)
    w4d = jnp.concatenate([edge_weight, jnp.zeros((pad,), jnp.float32)]
                          ).reshape(16, SUPER, SCH, CE)

    out_sc = _sc_propagate(src4d, dst4d, w4d, jnp.concatenate([x, h]))
    px = out_sc[0, :N]
    ph = out_sc[1, :N]

    w0x = jnp.transpose(Wx[:, 0], (1, 0, 2)).reshape(F, 512)
    w1x = jnp.transpose(Wx[:, 1], (1, 0, 2)).reshape(F, 512)
    w0h = jnp.transpose(Wh[:, 0], (1, 0, 2)).reshape(F, 512)
    w1h = jnp.transpose(Wh[:, 1], (1, 0, 2)).reshape(F, 512)
    ball = (bx + bh + bg).reshape(1, 512)
    lnp = jnp.stack([ln_h_g, ln_h_b, ln_c_g, ln_c_b, ln_o_g, ln_o_b])
    fc2p = jnp.pad(fc2_w, ((0, 0), (0, F - 1)))
    fc2b = jnp.pad(fc2_b, (0, F - 1)).reshape(1, F)

    pred, hid, cell = _tc_call(x, px, h, ph, c, w0x, w1x, w0h, w1h,
                               ball, wc, lnp, fc2p, fc2b)
    return pred[:, :1], hid[None], cell[None]


# speedup vs baseline: 9.6116x; 1.2825x over previous
import functools

import jax
import jax.numpy as jnp
from jax import lax
from jax.experimental import pallas as pl
from jax.experimental.pallas import tpu as pltpu
from jax.experimental.pallas import tpu_sc as plsc

N = 10000
N_PAD = 10240
F = 128
E = 320000
E_PAD = 327680
SUPER = 10
SCH = 32
CE = 64
NODES_PER_TILE = N_PAD // 16
RED = 128


def _sc_deg_body(dst_hbm, w_hbm, dinv_hbm,
                 dst_sc, w_sc, degp, tmp, dslice, partials):
    cid = lax.axis_index("c")
    sid = lax.axis_index("s")
    zeros16f = jnp.zeros((16,), jnp.float32)

    @pl.loop(0, N_PAD // 16)
    def _zero_deg(i):
        degp[pl.ds(i * 16, 16)] = zeros16f

    @pl.loop(0, SUPER)
    def _deg_super(sj):
        pltpu.sync_copy(dst_hbm.at[sid, sj], dst_sc)
        pltpu.sync_copy(w_hbm.at[sid, sj], w_sc)

        @pl.loop(0, SCH)
        def _deg(j):
            for k in range(CE // 16):
                dv = dst_sc[j, pl.ds(k * 16, 16)]
                wv = w_sc[j, pl.ds(k * 16, 16)]
                plsc.addupdate_scatter(degp, [dv], wv)

    pltpu.sync_copy(degp, partials.at[sid])
    plsc.subcore_barrier()

    @pl.loop(0, NODES_PER_TILE // RED)
    def _red(b):
        pltpu.sync_copy(
            partials.at[:, pl.ds(sid * NODES_PER_TILE + b * RED, RED)], tmp)

        @pl.loop(0, RED // 16)
        def _rsqrt(i):
            s = tmp[0, pl.ds(i * 16, 16)]
            for k in range(1, 16):
                s = s + tmp[k, pl.ds(i * 16, 16)]
            bits = plsc.bitcast(s, jnp.int32)
            y = plsc.bitcast(jnp.int32(0x5F3759DF) - (bits >> 1), jnp.float32)
            for _ in range(4):
                y = y * (1.5 - 0.5 * s * y * y)
            dslice[pl.ds(b * RED + i * 16, 16)] = jnp.where(s > 0.0, y, 0.0)

    @pl.when(cid == 0)
    def _write():
        pltpu.sync_copy(dslice,
                        dinv_hbm.at[pl.ds(sid * NODES_PER_TILE, NODES_PER_TILE)])


def _sc_deg(dst4d, w4d):
    mesh = plsc.VectorSubcoreMesh(core_axis_name="c", subcore_axis_name="s")
    return pl.kernel(
        _sc_deg_body,
        out_type=jax.ShapeDtypeStruct((N_PAD,), jnp.float32),
        mesh=mesh,
        scratch_types=[
            pltpu.VMEM((SCH, CE), jnp.int32),
            pltpu.VMEM((SCH, CE), jnp.float32),
            pltpu.VMEM((N_PAD,), jnp.float32),
            pltpu.VMEM((16, RED), jnp.float32),
            pltpu.VMEM((NODES_PER_TILE,), jnp.float32),
            pltpu.VMEM_SHARED((16, N_PAD), jnp.float32),
        ],
        compiler_params=pltpu.CompilerParams(needs_layout_passes=False),
    )(dst4d, w4d)


def _sc_propagate_body(src_hbm, dst_hbm, w_hbm, xh_hbm, dinv_hbm, out_hbm,
                       src_sc, dst_sc, w_sc, dsidx, dinv,
                       normA, normB, rowsA, rowsB, acc,
                       gsemA, gsemB, ssemA, ssemB):
    cid = lax.axis_index("c")
    sid = lax.axis_index("s")
    zeros16f = jnp.zeros((16,), jnp.float32)

    pltpu.sync_copy(dinv_hbm, dinv)

    @pl.loop(0, CE)
    def _zero_rows(r):
        for k in range(8):
            rowsA[r, pl.ds(k * 16, 16)] = zeros16f

    for b in range(NODES_PER_TILE // CE):
        pltpu.sync_copy(rowsA, acc.at[pl.ds(sid * NODES_PER_TILE + b * CE, CE), :])

    plsc.subcore_barrier()

    def norm_chunk(j, normc):
        for k in range(CE // 16):
            sv = src_sc[j, pl.ds(k * 16, 16)]
            dv = dst_sc[j, pl.ds(k * 16, 16)]
            wv = w_sc[j, pl.ds(k * 16, 16)]
            nv = -wv * plsc.load_gather(dinv, [sv]) * plsc.load_gather(dinv, [dv])
            normc[pl.ds(k * 16, 16)] = nv
            src_sc[j, pl.ds(k * 16, 16)] = sv + cid * N
            dsidx[j, pl.ds(k * 16, 16)] = dv

    def scale(rows, normc):
        @pl.loop(0, CE, unroll=8)
        def _scale(r):
            nv = plsc.load_gather(normc, [jnp.zeros((16,), jnp.int32) + r])
            for k in range(8):
                rows[r, pl.ds(k * 16, 16)] = rows[r, pl.ds(k * 16, 16)] * nv

    @pl.loop(0, SUPER * SCH // 2)
    def _pair(g):
        sj = g // (SCH // 2)
        j2 = (g % (SCH // 2)) * 2

        @pl.when(g % (SCH // 2) == 0)
        def _stage():
            pltpu.sync_copy(src_hbm.at[sid, sj], src_sc)
            pltpu.sync_copy(dst_hbm.at[sid, sj], dst_sc)
            pltpu.sync_copy(w_hbm.at[sid, sj], w_sc)

        norm_chunk(j2, normA)

        @pl.when(g > 0)
        def _wait_scatter_a():
            pltpu.make_async_copy(rowsA, acc.at[dsidx.at[j2]], ssemA).wait()

        pltpu.async_copy(xh_hbm.at[src_sc.at[j2]], rowsA, gsemA)
        norm_chunk(j2 + 1, normB)

        @pl.when(g > 0)
        def _wait_scatter_b():
            pltpu.make_async_copy(rowsB, acc.at[dsidx.at[j2 + 1]], ssemB).wait()

        pltpu.async_copy(xh_hbm.at[src_sc.at[j2 + 1]], rowsB, gsemB)
        pltpu.make_async_copy(xh_hbm.at[src_sc.at[j2]], rowsA, gsemA).wait()
        scale(rowsA, normA)
        pltpu.async_copy(rowsA, acc.at[dsidx.at[j2]], ssemA, add=True)
        pltpu.make_async_copy(xh_hbm.at[src_sc.at[j2 + 1]], rowsB, gsemB).wait()
        scale(rowsB, normB)
        pltpu.async_copy(rowsB, acc.at[dsidx.at[j2 + 1]], ssemB, add=True)

    pltpu.make_async_copy(rowsA, acc.at[dsidx.at[SCH - 2]], ssemA).wait()
    pltpu.make_async_copy(rowsB, acc.at[dsidx.at[SCH - 1]], ssemB).wait()
    plsc.subcore_barrier()

    pltpu.sync_copy(acc.at[pl.ds(sid * NODES_PER_TILE, NODES_PER_TILE), :],
                    out_hbm.at[cid, pl.ds(sid * NODES_PER_TILE, NODES_PER_TILE), :])


def _sc_propagate(src4d, dst4d, w4d, xh, dinv):
    mesh = plsc.VectorSubcoreMesh(core_axis_name="c", subcore_axis_name="s")
    return pl.kernel(
        _sc_propagate_body,
        out_type=jax.ShapeDtypeStruct((2, N_PAD, F), jnp.float32),
        mesh=mesh,
        scratch_types=[
            pltpu.VMEM((SCH, CE), jnp.int32),
            pltpu.VMEM((SCH, CE), jnp.int32),
            pltpu.VMEM((SCH, CE), jnp.float32),
            pltpu.VMEM((SCH, CE), jnp.int32),
            pltpu.VMEM((N_PAD,), jnp.float32),
            pltpu.VMEM((CE,), jnp.float32),
            pltpu.VMEM((CE,), jnp.float32),
            pltpu.VMEM((CE, F), jnp.float32),
            pltpu.VMEM((CE, F), jnp.float32),
            pltpu.VMEM_SHARED((N_PAD, F), jnp.float32),
            pltpu.SemaphoreType.DMA,
            pltpu.SemaphoreType.DMA,
            pltpu.SemaphoreType.DMA,
            pltpu.SemaphoreType.DMA,
        ],
        compiler_params=pltpu.CompilerParams(needs_layout_passes=False),
    )(src4d, dst4d, w4d, xh, dinv)


def _tc_body(x_r, px_r, h_r, ph_r, c_r, w0x_r, w1x_r, w0h_r, w1h_r,
             ball_r, wc_r, lnp_r, fc2w_r, fc2b_r, pred_r, hid_r, cell_r):
    x = x_r[...]
    px = px_r[...]
    h = h_r[...]
    ph = ph_r[...]
    c = c_r[...]
    pre = jnp.dot(x, w0x_r[...], preferred_element_type=jnp.float32)
    pre = pre + jnp.dot(px, w1x_r[...], preferred_element_type=jnp.float32)
    pre = pre + jnp.dot(h, w0h_r[...], preferred_element_type=jnp.float32)
    pre = pre + jnp.dot(ph, w1h_r[...], preferred_element_type=jnp.float32)
    pre = pre + ball_r[...]
    wc = wc_r[...]
    ii = pre[:, 0:128]
    ff = pre[:, 128:256]
    gg = pre[:, 256:384]
    oo = pre[:, 384:512]
    i = jax.nn.sigmoid(ii + wc[0:1] * c)
    f = jax.nn.sigmoid(ff + wc[1:2] * c)
    g = jnp.tanh(gg)
    cn = f * c + i * g
    o = jax.nn.sigmoid(oo + wc[2:3] * cn)
    hn = o * jnp.tanh(cn)

    def ln(v, gamma, beta):
        m = jnp.mean(v, axis=-1, keepdims=True)
        d = v - m
        var = jnp.mean(d * d, axis=-1, keepdims=True)
        return d * lax.rsqrt(var + 1e-5) * gamma + beta

    lnp = lnp_r[...]
    hid_r[...] = ln(hn, lnp[0:1], lnp[1:2])
    cell_r[...] = ln(cn, lnp[2:3], lnp[3:4])
    ov = ln(jnp.maximum(hn, 0.0), lnp[4:5], lnp[5:6])
    o2 = jnp.maximum(ov, 0.0)
    pred_r[...] = jax.nn.sigmoid(
        jnp.dot(o2, fc2w_r[...], preferred_element_type=jnp.float32) + fc2b_r[...])


def _tc_call(x, px, h, ph, c, w0x, w1x, w0h, w1h, ball, wc, lnp, fc2w, fc2b):
    R = 1000
    grid = (N // R,)
    row_spec = pl.BlockSpec((R, F), lambda i: (i, 0))
    full = lambda shape: pl.BlockSpec(shape, lambda i: tuple(0 for _ in shape))
    return pl.pallas_call(
        _tc_body,
        grid=grid,
        in_specs=[row_spec, row_spec, row_spec, row_spec, row_spec,
                  full((F, 512)), full((F, 512)), full((F, 512)), full((F, 512)),
                  full((1, 512)), full((3, F)), full((6, F)),
                  full((F, F)), full((1, F))],
        out_specs=[row_spec, row_spec, row_spec],
        out_shape=[jax.ShapeDtypeStruct((N, F), jnp.float32),
                   jax.ShapeDtypeStruct((N, F), jnp.float32),
                   jax.ShapeDtypeStruct((N, F), jnp.float32)],
    )(x, px, h, ph, c, w0x, w1x, w0h, w1h, ball, wc, lnp, fc2w, fc2b)


def kernel(X, edge_index, edge_weight, skip, H, C, Wx, bx, Wh, bh, wc, bg,
           ln_h_g, ln_h_b, ln_c_g, ln_c_b, ln_o_g, ln_o_b,
           fc1_w, fc1_b, fc2_w, fc2_b):
    x = X[0]
    h = H[0]
    c = C[0]
    src = edge_index[0]
    dst = edge_index[1]
    pad = E_PAD - E
    zi = jnp.zeros((pad,), jnp.int32)
    src4d = jnp.concatenate([src, zi]).reshape(16, SUPER, SCH, CE)
    dst4d = jnp.concatenate([dst, zi]).reshape(16, SUPER, SCH, CE)
    w4d = jnp.concatenate([edge_weight, jnp.zeros((pad,), jnp.float32)]
                          ).reshape(16, SUPER, SCH, CE)
    del pad

    dinv = _sc_deg(dst4d, w4d)
    out_sc = _sc_propagate(src4d, dst4d, w4d, jnp.concatenate([x, h]), dinv)
    px = out_sc[0, :N]
    ph = out_sc[1, :N]

    w0x = jnp.transpose(Wx[:, 0], (1, 0, 2)).reshape(F, 512)
    w1x = jnp.transpose(Wx[:, 1], (1, 0, 2)).reshape(F, 512)
    w0h = jnp.transpose(Wh[:, 0], (1, 0, 2)).reshape(F, 512)
    w1h = jnp.transpose(Wh[:, 1], (1, 0, 2)).reshape(F, 512)
    ball = (bx + bh + bg).reshape(1, 512)
    lnp = jnp.stack([ln_h_g, ln_h_b, ln_c_g, ln_c_b, ln_o_g, ln_o_b])
    fc2p = jnp.pad(fc2_w, ((0, 0), (0, F - 1)))
    fc2b = jnp.pad(fc2_b, (0, F - 1)).reshape(1, F)

    pred, hid, cell = _tc_call(x, px, h, ph, c, w0x, w1x, w0h, w1h,
                               ball, wc, lnp, fc2p, fc2b)
    return pred[:, :1], hid[None], cell[None]


# speedup vs baseline: 9.6489x; 1.0039x over previous
import functools

import jax
import jax.numpy as jnp
from jax import lax
from jax.experimental import pallas as pl
from jax.experimental.pallas import tpu as pltpu
from jax.experimental.pallas import tpu_sc as plsc

N = 10000
N_PAD = 10240
F = 128
E = 320000
E_PAD = 327680
SUPER = 10
SCH = 32
CE = 64
NODES_PER_TILE = N_PAD // 16
RED = 128


def _sc_deg_body(dst_hbm, w_hbm, dinv_hbm,
                 dst_sc, w_sc, degp, tmp, dslice, partials):
    cid = lax.axis_index("c")
    sid = lax.axis_index("s")
    zeros16f = jnp.zeros((16,), jnp.float32)

    @pl.loop(0, N_PAD // 16)
    def _zero_deg(i):
        degp[pl.ds(i * 16, 16)] = zeros16f

    @pl.loop(0, SUPER)
    def _deg_super(sj):
        pltpu.sync_copy(dst_hbm.at[sid, sj], dst_sc)
        pltpu.sync_copy(w_hbm.at[sid, sj], w_sc)

        @pl.loop(0, SCH)
        def _deg(j):
            for k in range(CE // 16):
                dv = dst_sc[j, pl.ds(k * 16, 16)]
                wv = w_sc[j, pl.ds(k * 16, 16)]
                plsc.addupdate_scatter(degp, [dv], wv)

    pltpu.sync_copy(degp, partials.at[sid])
    plsc.subcore_barrier()

    @pl.loop(0, NODES_PER_TILE // RED)
    def _red(b):
        pltpu.sync_copy(
            partials.at[:, pl.ds(sid * NODES_PER_TILE + b * RED, RED)], tmp)

        @pl.loop(0, RED // 16)
        def _rsqrt(i):
            s = tmp[0, pl.ds(i * 16, 16)]
            for k in range(1, 16):
                s = s + tmp[k, pl.ds(i * 16, 16)]
            bits = plsc.bitcast(s, jnp.int32)
            y = plsc.bitcast(jnp.int32(0x5F3759DF) - (bits >> 1), jnp.float32)
            for _ in range(4):
                y = y * (1.5 - 0.5 * s * y * y)
            dslice[pl.ds(b * RED + i * 16, 16)] = jnp.where(s > 0.0, y, 0.0)

    @pl.when(cid == 0)
    def _write():
        pltpu.sync_copy(dslice,
                        dinv_hbm.at[pl.ds(sid * NODES_PER_TILE, NODES_PER_TILE)])


def _sc_deg(dst4d, w4d):
    mesh = plsc.VectorSubcoreMesh(core_axis_name="c", subcore_axis_name="s")
    return pl.kernel(
        _sc_deg_body,
        out_type=jax.ShapeDtypeStruct((N_PAD,), jnp.float32),
        mesh=mesh,
        scratch_types=[
            pltpu.VMEM((SCH, CE), jnp.int32),
            pltpu.VMEM((SCH, CE), jnp.float32),
            pltpu.VMEM((N_PAD,), jnp.float32),
            pltpu.VMEM((16, RED), jnp.float32),
            pltpu.VMEM((NODES_PER_TILE,), jnp.float32),
            pltpu.VMEM_SHARED((16, N_PAD), jnp.float32),
        ],
        compiler_params=pltpu.CompilerParams(needs_layout_passes=False),
    )(dst4d, w4d)


def _sc_propagate_body(src_hbm, dst_hbm, w_hbm, xh_hbm, dinv_hbm, out_hbm,
                       src_sc, dst_sc, w_sc, dsidx, dinv,
                       normA, normB, rowsA, rowsB, acc,
                       gsemA, gsemB, ssemA, ssemB):
    cid = lax.axis_index("c")
    sid = lax.axis_index("s")
    zeros16f = jnp.zeros((16,), jnp.float32)

    pltpu.sync_copy(dinv_hbm, dinv)

    @pl.loop(0, CE)
    def _zero_rows(r):
        for k in range(8):
            rowsA[r, pl.ds(k * 16, 16)] = zeros16f

    for b in range(NODES_PER_TILE // CE):
        pltpu.sync_copy(rowsA, acc.at[pl.ds(sid * NODES_PER_TILE + b * CE, CE), :])

    plsc.subcore_barrier()

    def norm_chunk(j, normc):
        for k in range(CE // 16):
            sv = src_sc[j, pl.ds(k * 16, 16)]
            dv = dst_sc[j, pl.ds(k * 16, 16)]
            wv = w_sc[j, pl.ds(k * 16, 16)]
            nv = -wv * plsc.load_gather(dinv, [sv]) * plsc.load_gather(dinv, [dv])
            normc[pl.ds(k * 16, 16)] = nv
            src_sc[j, pl.ds(k * 16, 16)] = sv + cid * N
            dsidx[j, pl.ds(k * 16, 16)] = dv

    def scale(rows, normc):
        @pl.loop(0, CE, unroll=8)
        def _scale(r):
            nv = plsc.load_gather(normc, [jnp.zeros((16,), jnp.int32) + r])
            for k in range(8):
                rows[r, pl.ds(k * 16, 16)] = rows[r, pl.ds(k * 16, 16)] * nv

    def do_pair(j2, first):
        norm_chunk(j2, normA)
        if not first:
            pltpu.make_async_copy(rowsA, acc.at[dsidx.at[j2]], ssemA).wait()
        pltpu.async_copy(xh_hbm.at[src_sc.at[j2]], rowsA, gsemA)
        norm_chunk(j2 + 1, normB)
        if not first:
            pltpu.make_async_copy(rowsB, acc.at[dsidx.at[j2 + 1]], ssemB).wait()
        pltpu.async_copy(xh_hbm.at[src_sc.at[j2 + 1]], rowsB, gsemB)
        pltpu.make_async_copy(xh_hbm.at[src_sc.at[j2]], rowsA, gsemA).wait()
        scale(rowsA, normA)
        pltpu.async_copy(rowsA, acc.at[dsidx.at[j2]], ssemA, add=True)
        pltpu.make_async_copy(xh_hbm.at[src_sc.at[j2 + 1]], rowsB, gsemB).wait()
        scale(rowsB, normB)
        pltpu.async_copy(rowsB, acc.at[dsidx.at[j2 + 1]], ssemB, add=True)

    @pl.loop(0, SUPER)
    def _super(sj):
        pltpu.sync_copy(src_hbm.at[sid, sj], src_sc)
        pltpu.sync_copy(dst_hbm.at[sid, sj], dst_sc)
        pltpu.sync_copy(w_hbm.at[sid, sj], w_sc)

        @pl.when(sj == 0)
        def _first():
            do_pair(0, True)

        start = jnp.where(sj == 0, 1, 0)

        @pl.loop(start, SCH // 2)
        def _pairs(p):
            do_pair(p * 2, False)

    pltpu.make_async_copy(rowsA, acc.at[dsidx.at[SCH - 2]], ssemA).wait()
    pltpu.make_async_copy(rowsB, acc.at[dsidx.at[SCH - 1]], ssemB).wait()
    plsc.subcore_barrier()

    pltpu.sync_copy(acc.at[pl.ds(sid * NODES_PER_TILE, NODES_PER_TILE), :],
                    out_hbm.at[cid, pl.ds(sid * NODES_PER_TILE, NODES_PER_TILE), :])


def _sc_propagate(src4d, dst4d, w4d, xh, dinv):
    mesh = plsc.VectorSubcoreMesh(core_axis_name="c", subcore_axis_name="s")
    return pl.kernel(
        _sc_propagate_body,
        out_type=jax.ShapeDtypeStruct((2, N_PAD, F), jnp.float32),
        mesh=mesh,
        scratch_types=[
            pltpu.VMEM((SCH, CE), jnp.int32),
            pltpu.VMEM((SCH, CE), jnp.int32),
            pltpu.VMEM((SCH, CE), jnp.float32),
            pltpu.VMEM((SCH, CE), jnp.int32),
            pltpu.VMEM((N_PAD,), jnp.float32),
            pltpu.VMEM((CE,), jnp.float32),
            pltpu.VMEM((CE,), jnp.float32),
            pltpu.VMEM((CE, F), jnp.float32),
            pltpu.VMEM((CE, F), jnp.float32),
            pltpu.VMEM_SHARED((N_PAD, F), jnp.float32),
            pltpu.SemaphoreType.DMA,
            pltpu.SemaphoreType.DMA,
            pltpu.SemaphoreType.DMA,
            pltpu.SemaphoreType.DMA,
        ],
        compiler_params=pltpu.CompilerParams(needs_layout_passes=False),
    )(src4d, dst4d, w4d, xh, dinv)


def _tc_body(x_r, px_r, h_r, ph_r, c_r, w0x_r, w1x_r, w0h_r, w1h_r,
             ball_r, wc_r, lnp_r, fc2w_r, fc2b_r, pred_r, hid_r, cell_r):
    x = x_r[...]
    px = px_r[...]
    h = h_r[...]
    ph = ph_r[...]
    c = c_r[...]
    pre = jnp.dot(x, w0x_r[...], preferred_element_type=jnp.float32)
    pre = pre + jnp.dot(px, w1x_r[...], preferred_element_type=jnp.float32)
    pre = pre + jnp.dot(h, w0h_r[...], preferred_element_type=jnp.float32)
    pre = pre + jnp.dot(ph, w1h_r[...], preferred_element_type=jnp.float32)
    pre = pre + ball_r[...]
    wc = wc_r[...]
    ii = pre[:, 0:128]
    ff = pre[:, 128:256]
    gg = pre[:, 256:384]
    oo = pre[:, 384:512]
    i = jax.nn.sigmoid(ii + wc[0:1] * c)
    f = jax.nn.sigmoid(ff + wc[1:2] * c)
    g = jnp.tanh(gg)
    cn = f * c + i * g
    o = jax.nn.sigmoid(oo + wc[2:3] * cn)
    hn = o * jnp.tanh(cn)

    def ln(v, gamma, beta):
        m = jnp.mean(v, axis=-1, keepdims=True)
        d = v - m
        var = jnp.mean(d * d, axis=-1, keepdims=True)
        return d * lax.rsqrt(var + 1e-5) * gamma + beta

    lnp = lnp_r[...]
    hid_r[...] = ln(hn, lnp[0:1], lnp[1:2])
    cell_r[...] = ln(cn, lnp[2:3], lnp[3:4])
    ov = ln(jnp.maximum(hn, 0.0), lnp[4:5], lnp[5:6])
    o2 = jnp.maximum(ov, 0.0)
    pred_r[...] = jax.nn.sigmoid(
        jnp.dot(o2, fc2w_r[...], preferred_element_type=jnp.float32) + fc2b_r[...])


def _tc_call(x, px, h, ph, c, w0x, w1x, w0h, w1h, ball, wc, lnp, fc2w, fc2b):
    R = 1000
    grid = (N // R,)
    row_spec = pl.BlockSpec((R, F), lambda i: (i, 0))
    full = lambda shape: pl.BlockSpec(shape, lambda i: tuple(0 for _ in shape))
    return pl.pallas_call(
        _tc_body,
        grid=grid,
        in_specs=[row_spec, row_spec, row_spec, row_spec, row_spec,
                  full((F, 512)), full((F, 512)), full((F, 512)), full((F, 512)),
                  full((1, 512)), full((3, F)), full((6, F)),
                  full((F, F)), full((1, F))],
        out_specs=[row_spec, row_spec, row_spec],
        out_shape=[jax.ShapeDtypeStruct((N, F), jnp.float32),
                   jax.ShapeDtypeStruct((N, F), jnp.float32),
                   jax.ShapeDtypeStruct((N, F), jnp.float32)],
    )(x, px, h, ph, c, w0x, w1x, w0h, w1h, ball, wc, lnp, fc2w, fc2b)


def kernel(X, edge_index, edge_weight, skip, H, C, Wx, bx, Wh, bh, wc, bg,
           ln_h_g, ln_h_b, ln_c_g, ln_c_b, ln_o_g, ln_o_b,
           fc1_w, fc1_b, fc2_w, fc2_b):
    x = X[0]
    h = H[0]
    c = C[0]
    src = edge_index[0]
    dst = edge_index[1]
    pad = E_PAD - E
    zi = jnp.zeros((pad,), jnp.int32)
    src4d = jnp.concatenate([src, zi]).reshape(16, SUPER, SCH, CE)
    dst4d = jnp.concatenate([dst, zi]).reshape(16, SUPER, SCH, CE)
    w4d = jnp.concatenate([edge_weight, jnp.zeros((pad,), jnp.float32)]
                          ).reshape(16, SUPER, SCH, CE)
    del pad

    dinv = _sc_deg(dst4d, w4d)
    out_sc = _sc_propagate(src4d, dst4d, w4d, jnp.concatenate([x, h]), dinv)
    px = out_sc[0, :N]
    ph = out_sc[1, :N]

    w0x = jnp.transpose(Wx[:, 0], (1, 0, 2)).reshape(F, 512)
    w1x = jnp.transpose(Wx[:, 1], (1, 0, 2)).reshape(F, 512)
    w0h = jnp.transpose(Wh[:, 0], (1, 0, 2)).reshape(F, 512)
    w1h = jnp.transpose(Wh[:, 1], (1, 0, 2)).reshape(F, 512)
    ball = (bx + bh + bg).reshape(1, 512)
    lnp = jnp.stack([ln_h_g, ln_h_b, ln_c_g, ln_c_b, ln_o_g, ln_o_b])
    fc2p = jnp.pad(fc2_w, ((0, 0), (0, F - 1)))
    fc2b = jnp.pad(fc2_b, (0, F - 1)).reshape(1, F)

    pred, hid, cell = _tc_call(x, px, h, ph, c, w0x, w1x, w0h, w1h,
                               ball, wc, lnp, fc2p, fc2b)
    return pred[:, :1], hid[None], cell[None]


# speedup vs baseline: 10.6237x; 1.1010x over previous
import functools

import jax
import jax.numpy as jnp
from jax import lax
from jax.experimental import pallas as pl
from jax.experimental.pallas import tpu as pltpu
from jax.experimental.pallas import tpu_sc as plsc

N = 10000
N_PAD = 10240
F = 128
E = 320000
E_PAD = 327680
SUPER = 16
SCH = 20
CE = 64
NODES_PER_TILE = N_PAD // 16
RED = 128
N_ACC = 10048


def _sc_deg_body(src_hbm, dst_hbm, w_hbm, norm_hbm,
                 src_sc, dst_sc, w_sc, norm_sc, degp, tmp, dslice,
                 partials, sdinv):
    cid = lax.axis_index("c")
    sid = lax.axis_index("s")
    zeros16f = jnp.zeros((16,), jnp.float32)

    @pl.loop(0, N_PAD // 16)
    def _zero_deg(i):
        degp[pl.ds(i * 16, 16)] = zeros16f

    @pl.loop(0, SUPER)
    def _deg_super(sj):
        pltpu.sync_copy(dst_hbm.at[sid, sj], dst_sc)
        pltpu.sync_copy(w_hbm.at[sid, sj], w_sc)

        @pl.loop(0, SCH)
        def _deg(j):
            for k in range(CE // 16):
                dv = dst_sc[j, pl.ds(k * 16, 16)]
                wv = w_sc[j, pl.ds(k * 16, 16)]
                plsc.addupdate_scatter(degp, [dv], wv)

    pltpu.sync_copy(degp, partials.at[sid])
    plsc.subcore_barrier()

    @pl.loop(0, NODES_PER_TILE // RED)
    def _red(b):
        pltpu.sync_copy(
            partials.at[:, pl.ds(sid * NODES_PER_TILE + b * RED, RED)], tmp)

        @pl.loop(0, RED // 16)
        def _rsqrt(i):
            s = tmp[0, pl.ds(i * 16, 16)]
            for k in range(1, 16):
                s = s + tmp[k, pl.ds(i * 16, 16)]
            bits = plsc.bitcast(s, jnp.int32)
            y = plsc.bitcast(jnp.int32(0x5F3759DF) - (bits >> 1), jnp.float32)
            for _ in range(4):
                y = y * (1.5 - 0.5 * s * y * y)
            dslice[pl.ds(b * RED + i * 16, 16)] = jnp.where(s > 0.0, y, 0.0)

    pltpu.sync_copy(dslice, sdinv.at[pl.ds(sid * NODES_PER_TILE, NODES_PER_TILE)])
    plsc.subcore_barrier()
    pltpu.sync_copy(sdinv, degp)

    @pl.loop(0, SUPER // 2)
    def _norm_super(sj2):
        sj = sj2 * 2 + cid
        pltpu.sync_copy(src_hbm.at[sid, sj], src_sc)
        pltpu.sync_copy(dst_hbm.at[sid, sj], dst_sc)
        pltpu.sync_copy(w_hbm.at[sid, sj], w_sc)

        @pl.loop(0, SCH)
        def _nc(j):
            for k in range(CE // 16):
                sv = src_sc[j, pl.ds(k * 16, 16)]
                dv = dst_sc[j, pl.ds(k * 16, 16)]
                wv = w_sc[j, pl.ds(k * 16, 16)]
                nv = (-wv * plsc.load_gather(degp, [sv])
                      * plsc.load_gather(degp, [dv]))
                norm_sc[j, pl.ds(k * 16, 16)] = nv

        pltpu.sync_copy(norm_sc, norm_hbm.at[sid, sj])


def _sc_deg(src4d, dst4d, w4d):
    mesh = plsc.VectorSubcoreMesh(core_axis_name="c", subcore_axis_name="s")
    return pl.kernel(
        _sc_deg_body,
        out_type=jax.ShapeDtypeStruct((16, SUPER, SCH, CE), jnp.float32),
        mesh=mesh,
        scratch_types=[
            pltpu.VMEM((SCH, CE), jnp.int32),
            pltpu.VMEM((SCH, CE), jnp.int32),
            pltpu.VMEM((SCH, CE), jnp.float32),
            pltpu.VMEM((SCH, CE), jnp.float32),
            pltpu.VMEM((N_PAD,), jnp.float32),
            pltpu.VMEM((16, RED), jnp.float32),
            pltpu.VMEM((NODES_PER_TILE,), jnp.float32),
            pltpu.VMEM_SHARED((16, N_PAD), jnp.float32),
            pltpu.VMEM_SHARED((N_PAD,), jnp.float32),
        ],
        compiler_params=pltpu.CompilerParams(needs_layout_passes=False),
    )(src4d, dst4d, w4d)


def _sc_propagate_body(idx_hbm, dst_hbm, norm_hbm, xh_hbm, out_hbm,
                       src_sc, dst_sc, norm_sc, rows, acc,
                       gsem0, gsem1, gsem2, ssem0, ssem1):
    cid = lax.axis_index("c")
    sid = lax.axis_index("s")
    zeros16f = jnp.zeros((16,), jnp.float32)
    G = [rows.at[pl.ds(t * CE, CE), :] for t in range(3)]
    S = [rows.at[pl.ds((3 + t) * CE, CE), :] for t in range(2)]
    GS = [gsem0, gsem1, gsem2]
    SS = [ssem0, ssem1]

    @pl.loop(0, CE)
    def _zero_rows(r):
        for k in range(8):
            rows[r, pl.ds(k * 16, 16)] = zeros16f

    @pl.when(sid < 15)
    def _zero_acc():
        for b in range(NODES_PER_TILE // CE):
            pltpu.sync_copy(
                G[0], acc.at[pl.ds(sid * NODES_PER_TILE + b * CE, CE), :])

    @pl.when(sid == 15)
    def _zero_acc_last():
        for b in range((N_ACC - 15 * NODES_PER_TILE) // CE):
            pltpu.sync_copy(
                G[0], acc.at[pl.ds(15 * NODES_PER_TILE + b * CE, CE), :])

    plsc.subcore_barrier()

    def issue_gather(jj, tg):
        pltpu.async_copy(xh_hbm.at[src_sc.at[jj]], G[tg], GS[tg])

    def wait_gather(jj, tg):
        pltpu.make_async_copy(xh_hbm.at[src_sc.at[jj]], G[tg], GS[tg]).wait()

    def issue_scatter(jj, ts):
        pltpu.async_copy(S[ts], acc.at[dst_sc.at[jj]], SS[ts], add=True)

    def wait_scatter(jj, ts):
        pltpu.make_async_copy(S[ts], acc.at[dst_sc.at[jj]], SS[ts]).wait()

    def scale_into(jj, tg, ts):
        @pl.loop(0, CE, unroll=8)
        def _s(r):
            nv = plsc.load_gather(
                norm_sc, [jnp.zeros((16,), jnp.int32) + jj,
                          jnp.zeros((16,), jnp.int32) + r])
            for k in range(8):
                S[ts][r, pl.ds(k * 16, 16)] = G[tg][r, pl.ds(k * 16, 16)] * nv

    @pl.loop(0, SUPER)
    def _super(sj):
        pltpu.sync_copy(idx_hbm.at[cid, sid, sj], src_sc)
        pltpu.sync_copy(dst_hbm.at[sid, sj], dst_sc)
        pltpu.sync_copy(norm_hbm.at[sid, sj], norm_sc)

        for t in range(3):
            issue_gather(t, t)
        for j in range(2):
            wait_gather(j, j)
            scale_into(j, j, j)
            issue_scatter(j, j)
            issue_gather(j + 3, j)

        @pl.loop(0, (SCH - 2) // 6)
        def _grp(q):
            base = 2 + q * 6
            for i in range(6):
                jj = base + i
                tg = (2 + i) % 3
                ts = i % 2
                wait_gather(jj, tg)
                wait_scatter(jj, ts)
                scale_into(jj, tg, ts)
                issue_scatter(jj, ts)

                @pl.when(jj + 3 < SCH)
                def _ig():
                    issue_gather(jj + 3, tg)

        wait_scatter(0, 0)
        wait_scatter(1, 1)

    plsc.subcore_barrier()

    @pl.when(sid < 15)
    def _wr():
        pltpu.sync_copy(
            acc.at[pl.ds(sid * NODES_PER_TILE, NODES_PER_TILE), :],
            out_hbm.at[cid, pl.ds(sid * NODES_PER_TILE, NODES_PER_TILE), :])

    @pl.when(sid == 15)
    def _wr_last():
        pltpu.sync_copy(
            acc.at[pl.ds(15 * NODES_PER_TILE, N_ACC - 15 * NODES_PER_TILE), :],
            out_hbm.at[cid, pl.ds(15 * NODES_PER_TILE, N_ACC - 15 * NODES_PER_TILE), :])


def _sc_propagate(idx5d, dst4d, norm4d, xh):
    mesh = plsc.VectorSubcoreMesh(core_axis_name="c", subcore_axis_name="s")
    return pl.kernel(
        _sc_propagate_body,
        out_type=jax.ShapeDtypeStruct((2, N_ACC, F), jnp.float32),
        mesh=mesh,
        scratch_types=[
            pltpu.VMEM((SCH, CE), jnp.int32),
            pltpu.VMEM((SCH, CE), jnp.int32),
            pltpu.VMEM((SCH, CE), jnp.float32),
            pltpu.VMEM((5 * CE, F), jnp.float32),
            pltpu.VMEM_SHARED((N_ACC, F), jnp.float32),
            pltpu.SemaphoreType.DMA,
            pltpu.SemaphoreType.DMA,
            pltpu.SemaphoreType.DMA,
            pltpu.SemaphoreType.DMA,
            pltpu.SemaphoreType.DMA,
        ],
        compiler_params=pltpu.CompilerParams(needs_layout_passes=False),
    )(idx5d, dst4d, norm4d, xh)


def _tc_body(x_r, px_r, h_r, ph_r, c_r, w0x_r, w1x_r, w0h_r, w1h_r,
             ball_r, wc_r, lnp_r, fc2w_r, fc2b_r, pred_r, hid_r, cell_r):
    x = x_r[...]
    px = px_r[...]
    h = h_r[...]
    ph = ph_r[...]
    c = c_r[...]
    pre = jnp.dot(x, w0x_r[...], preferred_element_type=jnp.float32)
    pre = pre + jnp.dot(px, w1x_r[...], preferred_element_type=jnp.float32)
    pre = pre + jnp.dot(h, w0h_r[...], preferred_element_type=jnp.float32)
    pre = pre + jnp.dot(ph, w1h_r[...], preferred_element_type=jnp.float32)
    pre = pre + ball_r[...]
    wc = wc_r[...]
    ii = pre[:, 0:128]
    ff = pre[:, 128:256]
    gg = pre[:, 256:384]
    oo = pre[:, 384:512]
    i = jax.nn.sigmoid(ii + wc[0:1] * c)
    f = jax.nn.sigmoid(ff + wc[1:2] * c)
    g = jnp.tanh(gg)
    cn = f * c + i * g
    o = jax.nn.sigmoid(oo + wc[2:3] * cn)
    hn = o * jnp.tanh(cn)

    def ln(v, gamma, beta):
        m = jnp.mean(v, axis=-1, keepdims=True)
        d = v - m
        var = jnp.mean(d * d, axis=-1, keepdims=True)
        return d * lax.rsqrt(var + 1e-5) * gamma + beta

    lnp = lnp_r[...]
    hid_r[...] = ln(hn, lnp[0:1], lnp[1:2])
    cell_r[...] = ln(cn, lnp[2:3], lnp[3:4])
    ov = ln(jnp.maximum(hn, 0.0), lnp[4:5], lnp[5:6])
    o2 = jnp.maximum(ov, 0.0)
    pred_r[...] = jax.nn.sigmoid(
        jnp.dot(o2, fc2w_r[...], preferred_element_type=jnp.float32) + fc2b_r[...])


def _tc_call(x, px, h, ph, c, w0x, w1x, w0h, w1h, ball, wc, lnp, fc2w, fc2b):
    R = 1000
    grid = (N // R,)
    row_spec = pl.BlockSpec((R, F), lambda i: (i, 0))
    full = lambda shape: pl.BlockSpec(shape, lambda i: tuple(0 for _ in shape))
    return pl.pallas_call(
        _tc_body,
        grid=grid,
        in_specs=[row_spec, row_spec, row_spec, row_spec, row_spec,
                  full((F, 512)), full((F, 512)), full((F, 512)), full((F, 512)),
                  full((1, 512)), full((3, F)), full((6, F)),
                  full((F, F)), full((1, F))],
        out_specs=[row_spec, row_spec, row_spec],
        out_shape=[jax.ShapeDtypeStruct((N, F), jnp.float32),
                   jax.ShapeDtypeStruct((N, F), jnp.float32),
                   jax.ShapeDtypeStruct((N, F), jnp.float32)],
    )(x, px, h, ph, c, w0x, w1x, w0h, w1h, ball, wc, lnp, fc2w, fc2b)


def kernel(X, edge_index, edge_weight, skip, H, C, Wx, bx, Wh, bh, wc, bg,
           ln_h_g, ln_h_b, ln_c_g, ln_c_b, ln_o_g, ln_o_b,
           fc1_w, fc1_b, fc2_w, fc2_b):
    x = X[0]
    h = H[0]
    c = C[0]
    src = edge_index[0]
    dst = edge_index[1]
    pad = E_PAD - E
    zi = jnp.zeros((pad,), jnp.int32)
    src4d = jnp.concatenate([src, zi]).reshape(16, SUPER, SCH, CE)
    dst4d = jnp.concatenate([dst, zi]).reshape(16, SUPER, SCH, CE)
    w4d = jnp.concatenate([edge_weight, jnp.zeros((pad,), jnp.float32)]
                          ).reshape(16, SUPER, SCH, CE)
    del pad

    idx5d = jnp.stack([src4d, src4d + N])
    norm4d = _sc_deg(src4d, dst4d, w4d)
    out_sc = _sc_propagate(idx5d, dst4d, norm4d, jnp.concatenate([x, h]))
    px = out_sc[0, :N]
    ph = out_sc[1, :N]

    w0x = jnp.transpose(Wx[:, 0], (1, 0, 2)).reshape(F, 512)
    w1x = jnp.transpose(Wx[:, 1], (1, 0, 2)).reshape(F, 512)
    w0h = jnp.transpose(Wh[:, 0], (1, 0, 2)).reshape(F, 512)
    w1h = jnp.transpose(Wh[:, 1], (1, 0, 2)).reshape(F, 512)
    ball = (bx + bh + bg).reshape(1, 512)
    lnp = jnp.stack([ln_h_g, ln_h_b, ln_c_g, ln_c_b, ln_o_g, ln_o_b])
    fc2p = jnp.pad(fc2_w, ((0, 0), (0, F - 1)))
    fc2b = jnp.pad(fc2_b, (0, F - 1)).reshape(1, F)

    pred, hid, cell = _tc_call(x, px, h, ph, c, w0x, w1x, w0h, w1h,
                               ball, wc, lnp, fc2p, fc2b)
    return pred[:, :1], hid[None], cell[None]
